# baseline (speedup 1.0000x reference)
import jax
import jax.numpy as jnp
from jax import lax
from jax.experimental import pallas as pl
from jax.experimental.pallas import tpu as pltpu

N_TOK = 16384
HIDDEN = 2048
E = 16
K = 2
BT = 2048
GRID = N_TOK // BT


def _router_kernel(x_ref, w_ref, idx_ref, scr_ref, z_ref):
    i = pl.program_id(0)
    x = x_ref[...]
    w = w_ref[...]
    logits = lax.dot_general(
        w, x, (((1,), (1,)), ((), ())),
        preferred_element_type=jnp.float32,
    )

    part = jnp.sum(logits * logits)

    @pl.when(i == 0)
    def _():
        z_ref[0] = 0.0

    z_ref[0] += part

    iota = lax.broadcasted_iota(jnp.int32, (E, BT), 0)
    m1 = jnp.max(logits, axis=0, keepdims=True)
    i1 = jnp.min(jnp.where(logits == m1, iota, E), axis=0, keepdims=True)
    l2 = jnp.where(iota == i1, -jnp.inf, logits)
    m2 = jnp.max(l2, axis=0, keepdims=True)
    i2 = jnp.min(jnp.where(l2 == m2, iota, E), axis=0, keepdims=True)
    denom = jnp.sum(jnp.exp(logits - m1), axis=0, keepdims=True)
    s1 = 1.0 / denom
    s2 = jnp.exp(m2 - m1) / denom
    idx_ref[...] = jnp.concatenate([i1, i2], axis=0)
    scr_ref[...] = jnp.concatenate([s1, s2], axis=0)


def kernel(x, W):
    idx_t, scr_t, zsum = pl.pallas_call(
        _router_kernel,
        grid=(GRID,),
        in_specs=[
            pl.BlockSpec((BT, HIDDEN), lambda i: (i, 0)),
            pl.BlockSpec((E, HIDDEN), lambda i: (0, 0)),
        ],
        out_specs=[
            pl.BlockSpec((K, BT), lambda i: (0, i)),
            pl.BlockSpec((K, BT), lambda i: (0, i)),
            pl.BlockSpec(memory_space=pltpu.SMEM),
        ],
        out_shape=[
            jax.ShapeDtypeStruct((K, N_TOK), jnp.int32),
            jax.ShapeDtypeStruct((K, N_TOK), jnp.float32),
            jax.ShapeDtypeStruct((1,), jnp.float32),
        ],
    )(x, W)
    z_loss = zsum[0] / jnp.float32(N_TOK * E)
    aux_loss = jnp.zeros((), jnp.float32)
    return (idx_t.T, scr_t.T, aux_loss, z_loss)


# speedup vs baseline: 1.0306x; 1.0306x over previous
import jax
import jax.numpy as jnp
from jax import lax
from jax.experimental import pallas as pl
from jax.experimental.pallas import tpu as pltpu

N_TOK = 16384
HIDDEN = 2048
E = 16
K = 2
BT = 1024
GRID = N_TOK // BT


def _router_kernel(x_ref, w_ref, idx_ref, scr_ref, z_ref):
    i = pl.program_id(0)
    x = x_ref[...]
    w = w_ref[...]
    logits = lax.dot_general(
        w, x, (((1,), (1,)), ((), ())),
        preferred_element_type=jnp.float32,
    )

    part = jnp.sum(logits * logits)

    @pl.when(i == 0)
    def _():
        z_ref[0] = 0.0

    z_ref[0] += part

    iota = lax.broadcasted_iota(jnp.int32, (E, BT), 0)
    m1 = jnp.max(logits, axis=0, keepdims=True)
    i1 = jnp.min(jnp.where(logits == m1, iota, E), axis=0, keepdims=True)
    l2 = jnp.where(iota == i1, -jnp.inf, logits)
    m2 = jnp.max(l2, axis=0, keepdims=True)
    i2 = jnp.min(jnp.where(l2 == m2, iota, E), axis=0, keepdims=True)
    denom = jnp.sum(jnp.exp(logits - m1), axis=0, keepdims=True)
    s1 = 1.0 / denom
    s2 = jnp.exp(m2 - m1) / denom
    idx_ref[...] = jnp.concatenate([i1, i2], axis=0)
    scr_ref[...] = jnp.concatenate([s1, s2], axis=0)


def kernel(x, W):
    idx_t, scr_t, zsum = pl.pallas_call(
        _router_kernel,
        grid=(GRID,),
        in_specs=[
            pl.BlockSpec((BT, HIDDEN), lambda i: (i, 0)),
            pl.BlockSpec((E, HIDDEN), lambda i: (0, 0)),
        ],
        out_specs=[
            pl.BlockSpec((K, BT), lambda i: (0, i)),
            pl.BlockSpec((K, BT), lambda i: (0, i)),
            pl.BlockSpec(memory_space=pltpu.SMEM),
        ],
        out_shape=[
            jax.ShapeDtypeStruct((K, N_TOK), jnp.int32),
            jax.ShapeDtypeStruct((K, N_TOK), jnp.float32),
            jax.ShapeDtypeStruct((1,), jnp.float32),
        ],
    )(x, W)
    z_loss = zsum[0] / jnp.float32(N_TOK * E)
    aux_loss = jnp.zeros((), jnp.float32)
    return (idx_t.T, scr_t.T, aux_loss, z_loss)
